# Initial kernel scaffold; baseline (speedup 1.0000x reference)
import jax
import jax.numpy as jnp
from jax.experimental import pallas as pl


def kernel(hidden_states, gate_weight, secret_projection):
    raise NotImplementedError("write your pallas kernel here")


# speedup vs baseline: 5.3100x; 5.3100x over previous
import functools

import jax
import jax.numpy as jnp
from jax.experimental import pallas as pl
from jax.experimental.pallas import tpu as pltpu

_B, _S, _D, _E, _K = 4, 2048, 4096, 64, 8
_ALPHA = 0.1
_THRESH = 0.25
_TBLK = 512


def _router_kernel(x_ref, wc_ref, mask_ref, probs_ref, logits_ref):
    x = x_ref[...]
    both = jnp.dot(x, wc_ref[...], preferred_element_type=jnp.float32)
    raw = both[:, :_E]
    wat = both[:, _E:]

    inv_e = 1.0 / _E
    inv_em1 = 1.0 / (_E - 1)

    r_mean = jnp.sum(raw, axis=1, keepdims=True) * inv_e
    r_var = jnp.sum((raw - r_mean) ** 2, axis=1, keepdims=True) * inv_em1
    logits_std = jnp.sqrt(r_var) + 1e-6

    w_mean = jnp.sum(wat, axis=1, keepdims=True) * inv_e
    w_var = jnp.sum((wat - w_mean) ** 2, axis=1, keepdims=True) * inv_em1
    w_std = jnp.sqrt(w_var) + 1e-6
    wat_norm = (wat - w_mean) / w_std

    m1 = jnp.max(raw, axis=1, keepdims=True)
    ex = jnp.exp(raw - m1)
    sumex = jnp.sum(ex, axis=1, keepdims=True)
    probs = ex / sumex
    idx = jax.lax.broadcasted_iota(jnp.int32, raw.shape, 1)
    p1 = jnp.max(probs, axis=1, keepdims=True)
    is_p1 = probs == p1
    first1 = jnp.min(jnp.where(is_p1, idx, _E), axis=1, keepdims=True)
    p2 = jnp.max(jnp.where(idx == first1, -1.0, probs), axis=1, keepdims=True)
    gap = p1 - p2
    gate = jax.nn.sigmoid(10.0 * (_THRESH - gap))

    injection = gate * (_ALPHA * logits_std) * wat_norm
    max_noise = logits_std * 1.5
    injection = jnp.clip(injection, -max_noise, max_noise)
    final = raw + injection

    sel = jnp.zeros(final.shape, dtype=jnp.bool_)
    work = final
    neg_inf = jnp.float32(-jnp.inf)
    for _ in range(_K):
        mk = jnp.max(work, axis=1, keepdims=True)
        cand = work == mk
        fidx = jnp.min(jnp.where(cand, idx, _E), axis=1, keepdims=True)
        chosen = idx == fidx
        sel = jnp.logical_or(sel, chosen)
        work = jnp.where(chosen, neg_inf, work)

    fmax = jnp.max(final, axis=1, keepdims=True)
    fe = jnp.where(sel, jnp.exp(final - fmax), 0.0)
    fz = jnp.sum(fe, axis=1, keepdims=True)
    w = fe / fz

    mask_ref[...] = jnp.where(sel, w, 0.0)
    probs_ref[...] = jnp.sum(w, axis=1, keepdims=True)
    logits_ref[...] = jnp.where(sel, jnp.log(w + 1e-9), neg_inf)


def kernel(hidden_states, gate_weight, secret_projection):
    n = _B * _S
    x = hidden_states.reshape(n, _D)
    wc = jnp.concatenate((gate_weight, secret_projection), axis=1)
    grid = (n // _TBLK,)
    mask, probs, logits = pl.pallas_call(
        _router_kernel,
        grid=grid,
        in_specs=[
            pl.BlockSpec((_TBLK, _D), lambda i: (i, 0)),
            pl.BlockSpec((_D, 2 * _E), lambda i: (0, 0)),
        ],
        out_specs=[
            pl.BlockSpec((_TBLK, _E), lambda i: (i, 0)),
            pl.BlockSpec((_TBLK, 1), lambda i: (i, 0)),
            pl.BlockSpec((_TBLK, _E), lambda i: (i, 0)),
        ],
        out_shape=[
            jax.ShapeDtypeStruct((n, _E), jnp.float32),
            jax.ShapeDtypeStruct((n, 1), jnp.float32),
            jax.ShapeDtypeStruct((n, _E), jnp.float32),
        ],
        compiler_params=pltpu.CompilerParams(
            dimension_semantics=("arbitrary",),
        ),
    )(x, wc)
    return (
        mask.reshape(_B, _S, _E),
        probs.reshape(_B, _S, 1),
        logits.reshape(_B, _S, _E),
    )


# speedup vs baseline: 9.4819x; 1.7857x over previous
import functools

import jax
import jax.numpy as jnp
from jax.experimental import pallas as pl
from jax.experimental.pallas import tpu as pltpu

_B, _S, _D, _E, _K = 4, 2048, 4096, 64, 8
_ALPHA = 0.1
_THRESH = 0.25
_TBLK = 512


def _router_kernel(x_ref, wc_ref, mask_ref, probs_ref, logits_ref):
    x = x_ref[...]
    wc = wc_ref[...]
    both = jax.lax.dot_general(
        wc, x, (((0,), (1,)), ((), ())), preferred_element_type=jnp.float32
    )
    raw = both[:_E, :]
    wat = both[_E:, :]

    inv_e = 1.0 / _E
    inv_em1 = 1.0 / (_E - 1)

    r_mean = jnp.sum(raw, axis=0, keepdims=True) * inv_e
    r_var = jnp.sum((raw - r_mean) ** 2, axis=0, keepdims=True) * inv_em1
    logits_std = jnp.sqrt(r_var) + 1e-6

    w_mean = jnp.sum(wat, axis=0, keepdims=True) * inv_e
    w_var = jnp.sum((wat - w_mean) ** 2, axis=0, keepdims=True) * inv_em1
    w_std = jnp.sqrt(w_var) + 1e-6
    wat_norm = (wat - w_mean) / w_std

    m1 = jnp.max(raw, axis=0, keepdims=True)
    ex = jnp.exp(raw - m1)
    sumex = jnp.sum(ex, axis=0, keepdims=True)
    idx = jax.lax.broadcasted_iota(jnp.int32, raw.shape, 0)
    is_p1 = ex == 1.0
    first1 = jnp.min(jnp.where(is_p1, idx, _E), axis=0, keepdims=True)
    ex2 = jnp.max(jnp.where(idx == first1, -1.0, ex), axis=0, keepdims=True)
    gap = (1.0 - ex2) / sumex
    gate = jax.nn.sigmoid(10.0 * (_THRESH - gap))

    injection = gate * (_ALPHA * logits_std) * wat_norm
    max_noise = logits_std * 1.5
    injection = jnp.clip(injection, -max_noise, max_noise)
    final = raw + injection

    sel = jnp.zeros(final.shape, dtype=jnp.bool_)
    work = final
    neg_inf = jnp.float32(-jnp.inf)
    for _ in range(_K):
        mk = jnp.max(work, axis=0, keepdims=True)
        cand = work == mk
        fidx = jnp.min(jnp.where(cand, idx, _E), axis=0, keepdims=True)
        chosen = idx == fidx
        sel = jnp.logical_or(sel, chosen)
        work = jnp.where(chosen, neg_inf, work)

    fmax = jnp.max(final, axis=0, keepdims=True)
    fe = jnp.where(sel, jnp.exp(final - fmax), 0.0)
    fz = jnp.sum(fe, axis=0, keepdims=True)
    w = fe / fz

    mask_ref[...] = jnp.transpose(jnp.where(sel, w, 0.0))
    probs_ref[...] = jnp.transpose(jnp.sum(w, axis=0, keepdims=True))
    logits_ref[...] = jnp.transpose(jnp.where(sel, jnp.log(w + 1e-9), neg_inf))


def kernel(hidden_states, gate_weight, secret_projection):
    n = _B * _S
    x = hidden_states.reshape(n, _D)
    wc = jnp.concatenate((gate_weight, secret_projection), axis=1)
    grid = (n // _TBLK,)
    mask, probs, logits = pl.pallas_call(
        _router_kernel,
        grid=grid,
        in_specs=[
            pl.BlockSpec((_TBLK, _D), lambda i: (i, 0)),
            pl.BlockSpec((_D, 2 * _E), lambda i: (0, 0)),
        ],
        out_specs=[
            pl.BlockSpec((_TBLK, _E), lambda i: (i, 0)),
            pl.BlockSpec((_TBLK, 1), lambda i: (i, 0)),
            pl.BlockSpec((_TBLK, _E), lambda i: (i, 0)),
        ],
        out_shape=[
            jax.ShapeDtypeStruct((n, _E), jnp.float32),
            jax.ShapeDtypeStruct((n, 1), jnp.float32),
            jax.ShapeDtypeStruct((n, _E), jnp.float32),
        ],
        compiler_params=pltpu.CompilerParams(
            dimension_semantics=("arbitrary",),
        ),
    )(x, wc)
    return (
        mask.reshape(_B, _S, _E),
        probs.reshape(_B, _S, 1),
        logits.reshape(_B, _S, _E),
    )
